# baseline (speedup 1.0000x reference)
import jax
import jax.numpy as jnp
from jax.experimental import pallas as pl

N = 50000
E = 800000
G = 256
HEADS = 4
HID = 16
OUT = 64


def _gat_layer(x, src, dst, W, a_src, a_dst, b, heads, out_dim, concat):
    n = x.shape[0]
    h = (x @ W).reshape(n, heads, out_dim)
    alpha_src = (h * a_src[None]).sum(-1)
    alpha_dst = (h * a_dst[None]).sum(-1)
    e = alpha_src[src] + alpha_dst[dst]
    e = jax.nn.leaky_relu(e, 0.2)
    e_max = jax.ops.segment_max(e, dst, num_segments=n)
    e_max = jnp.where(jnp.isfinite(e_max), e_max, 0.0)
    ex = jnp.exp(e - e_max[dst])
    denom = jax.ops.segment_sum(ex, dst, num_segments=n)
    alpha = ex / (denom[dst] + 1e-16)
    msg = h[src] * alpha[:, :, None]
    out = jax.ops.segment_sum(msg, dst, num_segments=n)
    if concat:
        out = out.reshape(n, heads * out_dim)
    else:
        out = out.mean(axis=1)
    return out + b


def kernel(x, edge_index, batch, W1, a_src1, a_dst1, b1, W2, a_src2, a_dst2, b2, W3, a_src3, a_dst3, b3):
    n = x.shape[0]
    loop = jnp.arange(n, dtype=edge_index.dtype)
    src = jnp.concatenate([edge_index[0], loop])
    dst = jnp.concatenate([edge_index[1], loop])
    h = _gat_layer(x, src, dst, W1, a_src1, a_dst1, b1, HEADS, HID, True)
    h = jax.nn.relu(h)
    h = _gat_layer(h, src, dst, W2, a_src2, a_dst2, b2, HEADS, HID, True)
    h = jax.nn.relu(h)
    h = _gat_layer(h, src, dst, W3, a_src3, a_dst3, b3, 1, OUT, False)
    sums = jax.ops.segment_sum(h, batch, num_segments=G)
    cnt = jax.ops.segment_sum(jnp.ones((n,), dtype=h.dtype), batch, num_segments=G)
    return sums / jnp.maximum(cnt, 1.0)[:, None]


# speedup vs baseline: 3.2412x; 3.2412x over previous
import functools

import jax
import jax.numpy as jnp
from jax.experimental import pallas as pl

_N = 50000
_E = 800000
_G = 256
_HEADS = 4
_HID = 16
_OUT = 64

_NODE_BLK = 1000
_EDGE_BLK = 1000


def _layer_body(x_ref, w_ref, asrc_ref, adst_ref, b_ref, h_ref, es_ref, ed_ref,
                *, act):
    x = x_ref[...]
    if act:
        x = jnp.maximum(x + b_ref[...], 0.0)
    h = jnp.dot(x, w_ref[...], preferred_element_type=jnp.float32)
    h_ref[...] = h
    es_ref[...] = jnp.dot(h, asrc_ref[...], preferred_element_type=jnp.float32)
    ed_ref[...] = jnp.dot(h, adst_ref[...], preferred_element_type=jnp.float32)


def _layer(x, W, a_src, a_dst, b_prev, act, heads):
    n, din = x.shape
    dout = W.shape[1]
    hid = dout // heads
    proj_s = jnp.zeros((dout, heads), jnp.float32)
    proj_d = jnp.zeros((dout, heads), jnp.float32)
    rows = jnp.arange(dout)
    proj_s = proj_s.at[rows, rows // hid].set(a_src.reshape(-1))
    proj_d = proj_d.at[rows, rows // hid].set(a_dst.reshape(-1))
    grid = (n // _NODE_BLK,)
    h, es, ed = pl.pallas_call(
        functools.partial(_layer_body, act=act),
        grid=grid,
        in_specs=[
            pl.BlockSpec((_NODE_BLK, din), lambda i: (i, 0)),
            pl.BlockSpec((din, dout), lambda i: (0, 0)),
            pl.BlockSpec((dout, heads), lambda i: (0, 0)),
            pl.BlockSpec((dout, heads), lambda i: (0, 0)),
            pl.BlockSpec((1, din), lambda i: (0, 0)),
        ],
        out_specs=[
            pl.BlockSpec((_NODE_BLK, dout), lambda i: (i, 0)),
            pl.BlockSpec((_NODE_BLK, heads), lambda i: (i, 0)),
            pl.BlockSpec((_NODE_BLK, heads), lambda i: (i, 0)),
        ],
        out_shape=[
            jax.ShapeDtypeStruct((n, dout), jnp.float32),
            jax.ShapeDtypeStruct((n, heads), jnp.float32),
            jax.ShapeDtypeStruct((n, heads), jnp.float32),
        ],
    )(x, W, proj_s, proj_d, b_prev.reshape(1, -1))
    return h, es, ed


def _edge_e_body(es_ref, ed_ref, o_ref):
    e = es_ref[...] + ed_ref[...]
    o_ref[...] = jnp.where(e >= 0.0, e, 0.2 * e)


def _edge_ex_body(e_ref, m_ref, o_ref):
    o_ref[...] = jnp.exp(e_ref[...] - m_ref[...])


def _edge_msg_body(h_ref, ex_ref, den_ref, o_ref, *, heads):
    blk = h_ref.shape[0]
    alpha = ex_ref[...] / (den_ref[...] + 1e-16)
    h = h_ref[...].reshape(blk, heads, -1)
    o_ref[...] = (h * alpha[:, :, None]).reshape(blk, -1)


def _edge_call(body, args, out_dim):
    m = args[0].shape[0]
    grid = (m // _EDGE_BLK,)
    specs = [pl.BlockSpec((_EDGE_BLK, a.shape[1]), lambda i: (i, 0)) for a in args]
    return pl.pallas_call(
        body,
        grid=grid,
        in_specs=specs,
        out_specs=pl.BlockSpec((_EDGE_BLK, out_dim), lambda i: (i, 0)),
        out_shape=jax.ShapeDtypeStruct((m, out_dim), jnp.float32),
    )(*args)


def _pool_body(s_ref, c_ref, o_ref):
    o_ref[...] = s_ref[...] / jnp.maximum(c_ref[...], 1.0)


def _gat_layer(x, src, dst, W, a_src, a_dst, b_prev, act, heads):
    n = x.shape[0]
    h, es, ed = _layer(x, W, a_src, a_dst, b_prev, act, heads)
    e = _edge_call(_edge_e_body, (es[src], ed[dst]), heads)
    e_max = jax.ops.segment_max(e, dst, num_segments=n)
    e_max = jnp.where(jnp.isfinite(e_max), e_max, 0.0)
    ex = _edge_call(_edge_ex_body, (e, e_max[dst]), heads)
    denom = jax.ops.segment_sum(ex, dst, num_segments=n)
    msg = _edge_call(
        functools.partial(_edge_msg_body, heads=heads),
        (h[src], ex, denom[dst]), h.shape[1])
    return jax.ops.segment_sum(msg, dst, num_segments=n)


def kernel(x, edge_index, batch, W1, a_src1, a_dst1, b1,
           W2, a_src2, a_dst2, b2, W3, a_src3, a_dst3, b3):
    n = x.shape[0]
    loop = jnp.arange(n, dtype=edge_index.dtype)
    src = jnp.concatenate([edge_index[0], loop])
    dst = jnp.concatenate([edge_index[1], loop])
    zero_b = jnp.zeros((x.shape[1],), jnp.float32)
    o1 = _gat_layer(x, src, dst, W1, a_src1, a_dst1, zero_b, False, _HEADS)
    o2 = _gat_layer(o1, src, dst, W2, a_src2, a_dst2, b1, True, _HEADS)
    o3 = _gat_layer(o2, src, dst, W3, a_src3, a_dst3, b2, True, 1)
    h3 = o3 + b3[None, :]
    sums = jax.ops.segment_sum(h3, batch, num_segments=_G)
    cnt = jax.ops.segment_sum(jnp.ones((n, 1), jnp.float32), batch,
                              num_segments=_G)
    return pl.pallas_call(
        _pool_body,
        grid=(1,),
        in_specs=[
            pl.BlockSpec((_G, _OUT), lambda i: (0, 0)),
            pl.BlockSpec((_G, 1), lambda i: (0, 0)),
        ],
        out_specs=pl.BlockSpec((_G, _OUT), lambda i: (0, 0)),
        out_shape=jax.ShapeDtypeStruct((_G, _OUT), jnp.float32),
    )(sums, cnt)


# speedup vs baseline: 3.3420x; 1.0311x over previous
import functools

import jax
import jax.numpy as jnp
from jax.experimental import pallas as pl

_N = 50000
_E = 800000
_G = 256
_HEADS = 4
_HID = 16
_OUT = 64

_NODE_BLK = 1000
_EDGE_BLK = 5000


def _layer_body(x_ref, w_ref, asrc_ref, adst_ref, b_ref, h_ref, es_ref, ed_ref,
                *, act):
    x = x_ref[...]
    if act:
        x = jnp.maximum(x + b_ref[...], 0.0)
    h = jnp.dot(x, w_ref[...], preferred_element_type=jnp.float32)
    h_ref[...] = h
    es_ref[...] = jnp.dot(h, asrc_ref[...], preferred_element_type=jnp.float32)
    ed_ref[...] = jnp.dot(h, adst_ref[...], preferred_element_type=jnp.float32)


def _layer(x, W, a_src, a_dst, b_prev, act, heads):
    n, din = x.shape
    dout = W.shape[1]
    hid = dout // heads
    proj_s = jnp.zeros((dout, heads), jnp.float32)
    proj_d = jnp.zeros((dout, heads), jnp.float32)
    rows = jnp.arange(dout)
    proj_s = proj_s.at[rows, rows // hid].set(a_src.reshape(-1))
    proj_d = proj_d.at[rows, rows // hid].set(a_dst.reshape(-1))
    grid = (n // _NODE_BLK,)
    h, es, ed = pl.pallas_call(
        functools.partial(_layer_body, act=act),
        grid=grid,
        in_specs=[
            pl.BlockSpec((_NODE_BLK, din), lambda i: (i, 0)),
            pl.BlockSpec((din, dout), lambda i: (0, 0)),
            pl.BlockSpec((dout, heads), lambda i: (0, 0)),
            pl.BlockSpec((dout, heads), lambda i: (0, 0)),
            pl.BlockSpec((1, din), lambda i: (0, 0)),
        ],
        out_specs=[
            pl.BlockSpec((_NODE_BLK, dout), lambda i: (i, 0)),
            pl.BlockSpec((_NODE_BLK, heads), lambda i: (i, 0)),
            pl.BlockSpec((_NODE_BLK, heads), lambda i: (i, 0)),
        ],
        out_shape=[
            jax.ShapeDtypeStruct((n, dout), jnp.float32),
            jax.ShapeDtypeStruct((n, heads), jnp.float32),
            jax.ShapeDtypeStruct((n, heads), jnp.float32),
        ],
    )(x, W, proj_s, proj_d, b_prev.reshape(1, -1))
    return h, es, ed


def _edge_e_body(es_ref, ed_ref, o_ref):
    e = es_ref[...] + ed_ref[...]
    o_ref[...] = jnp.where(e >= 0.0, e, 0.2 * e)


def _edge_ex_body(e_ref, m_ref, o_ref):
    o_ref[...] = jnp.exp(e_ref[...] - m_ref[...])


def _edge_msg_body(h_ref, ex_ref, den_ref, o_ref, *, heads):
    blk = h_ref.shape[0]
    alpha = ex_ref[...] / (den_ref[...] + 1e-16)
    h = h_ref[...].reshape(blk, heads, -1)
    o_ref[...] = (h * alpha[:, :, None]).reshape(blk, -1)


def _edge_call(body, args, out_dim):
    m = args[0].shape[0]
    grid = (m // _EDGE_BLK,)
    specs = [pl.BlockSpec((_EDGE_BLK, a.shape[1]), lambda i: (i, 0)) for a in args]
    return pl.pallas_call(
        body,
        grid=grid,
        in_specs=specs,
        out_specs=pl.BlockSpec((_EDGE_BLK, out_dim), lambda i: (i, 0)),
        out_shape=jax.ShapeDtypeStruct((m, out_dim), jnp.float32),
    )(*args)


def _pool_body(s_ref, c_ref, o_ref):
    o_ref[...] = s_ref[...] / jnp.maximum(c_ref[...], 1.0)


def _gat_layer(x, src, dst, W, a_src, a_dst, b_prev, act, heads):
    n = x.shape[0]
    h, es, ed = _layer(x, W, a_src, a_dst, b_prev, act, heads)
    e = _edge_call(_edge_e_body, (es[src], ed[dst]), heads)
    e_max = jax.ops.segment_max(e, dst, num_segments=n)
    e_max = jnp.where(jnp.isfinite(e_max), e_max, 0.0)
    ex = _edge_call(_edge_ex_body, (e, e_max[dst]), heads)
    denom = jax.ops.segment_sum(ex, dst, num_segments=n)
    msg = _edge_call(
        functools.partial(_edge_msg_body, heads=heads),
        (h[src], ex, denom[dst]), h.shape[1])
    return jax.ops.segment_sum(msg, dst, num_segments=n)


def kernel(x, edge_index, batch, W1, a_src1, a_dst1, b1,
           W2, a_src2, a_dst2, b2, W3, a_src3, a_dst3, b3):
    n = x.shape[0]
    loop = jnp.arange(n, dtype=edge_index.dtype)
    src = jnp.concatenate([edge_index[0], loop])
    dst = jnp.concatenate([edge_index[1], loop])
    zero_b = jnp.zeros((x.shape[1],), jnp.float32)
    o1 = _gat_layer(x, src, dst, W1, a_src1, a_dst1, zero_b, False, _HEADS)
    o2 = _gat_layer(o1, src, dst, W2, a_src2, a_dst2, b1, True, _HEADS)
    o3 = _gat_layer(o2, src, dst, W3, a_src3, a_dst3, b2, True, 1)
    h3 = o3 + b3[None, :]
    sums = jax.ops.segment_sum(h3, batch, num_segments=_G)
    cnt = jax.ops.segment_sum(jnp.ones((n, 1), jnp.float32), batch,
                              num_segments=_G)
    return pl.pallas_call(
        _pool_body,
        grid=(1,),
        in_specs=[
            pl.BlockSpec((_G, _OUT), lambda i: (0, 0)),
            pl.BlockSpec((_G, 1), lambda i: (0, 0)),
        ],
        out_specs=pl.BlockSpec((_G, _OUT), lambda i: (0, 0)),
        out_shape=jax.ShapeDtypeStruct((_G, _OUT), jnp.float32),
    )(sums, cnt)
